# baseline (speedup 1.0000x reference)
import functools

import jax
import jax.numpy as jnp
from jax import lax
from jax.experimental import pallas as pl
from jax.experimental.pallas import tpu as pltpu
from jax.experimental.pallas import tpu_sc as plsc

NC, NS = 2, 16
NW = NC * NS


NBUF = 2


def _layout_edges(src, dst, n):
    e = src.shape[0]
    ck = 128
    ch = -(-e // (NW * ck))
    ch = NBUF * (-(-ch // NBUF))
    pad = NW * ch * ck - e
    if pad:
        _, npad = _row_split(n)
        ndump = _acc_rows(n, npad) - n
        i = jnp.arange(pad, dtype=jnp.int32)
        src = jnp.concatenate([src, (i * 131) % n])
        dst = jnp.concatenate([dst, n + i % ndump])
    return src.reshape(NW, ch, ck), dst.reshape(NW, ch, ck), ch, ck


def _row_split(n):
    rpt = 8 * (-(-n // (8 * NS)))
    npad = rpt * NS
    return rpt, npad


def _acc_rows(n, npad):
    return npad if npad > n else npad + 8


def _deg_kernel(n, d, ch, ck):
    rpt, npad = _row_split(n)
    nacc = _acc_rows(n, npad)
    mesh = plsc.VectorSubcoreMesh(core_axis_name="c", subcore_axis_name="s")

    @functools.partial(
        pl.kernel,
        out_type=jax.ShapeDtypeStruct((NC, npad, d), jnp.float32),
        mesh=mesh,
        scratch_types=[
            pltpu.VMEM((ch, ck), jnp.int32),
            pltpu.VMEM((ck, d), jnp.float32),
            pltpu.VMEM_SHARED((nacc, d), jnp.float32),
        ],
    )
    def deg_k(dst_hbm, ones_hbm, z_hbm, out_hbm, idx_v, ones_v, acc_sh):
        cid = lax.axis_index("c")
        sid = lax.axis_index("s")
        wid = sid * NC + cid
        r0 = sid * rpt
        pltpu.sync_copy(z_hbm.at[pl.ds(r0, rpt)], acc_sh.at[pl.ds(r0, rpt)])
        pltpu.sync_copy(dst_hbm.at[wid], idx_v)
        pltpu.sync_copy(ones_hbm, ones_v)
        plsc.subcore_barrier()

        @pl.loop(0, ch)
        def _(c):
            pltpu.sync_copy(ones_v, acc_sh.at[idx_v.at[c]], add=True)

        plsc.subcore_barrier()
        pltpu.sync_copy(acc_sh.at[pl.ds(r0, rpt)],
                        out_hbm.at[cid, pl.ds(r0, rpt)])

    return deg_k


def _msg_kernel(n, d, ch, ck):
    rpt, npad = _row_split(n)
    nacc = _acc_rows(n, npad)
    mesh = plsc.VectorSubcoreMesh(core_axis_name="c", subcore_axis_name="s")

    @functools.partial(
        pl.kernel,
        out_type=jax.ShapeDtypeStruct((NC, npad, d), jnp.float32),
        mesh=mesh,
        scratch_types=[pltpu.VMEM((ch, ck), jnp.int32)]
          + [pltpu.VMEM((ck, d), jnp.float32) for _ in range(NBUF)]
          + [pltpu.VMEM((1, ck), jnp.int32) for _ in range(NBUF)]
          + [pltpu.VMEM_SHARED((nacc, d), jnp.float32)]
          + [pltpu.SemaphoreType.DMA for _ in range(3 * NBUF)],
    )
    def msg_k(xs_hbm, src_hbm, dst_hbm, z_hbm, out_hbm, si_v, *rest):
        rows = rest[:NBUF]
        dslot = rest[NBUF:2 * NBUF]
        acc_sh = rest[2 * NBUF]
        rsem = rest[2 * NBUF + 1:2 * NBUF + 1 + NBUF]
        dsem = rest[2 * NBUF + 1 + NBUF:2 * NBUF + 1 + 2 * NBUF]
        ssem = rest[2 * NBUF + 1 + 2 * NBUF:]
        cid = lax.axis_index("c")
        sid = lax.axis_index("s")
        wid = sid * NC + cid
        r0 = sid * rpt
        pltpu.sync_copy(z_hbm.at[pl.ds(r0, rpt)], acc_sh.at[pl.ds(r0, rpt)])
        pltpu.sync_copy(src_hbm.at[wid], si_v)
        plsc.subcore_barrier()

        for b in range(NBUF):
            pltpu.async_copy(dst_hbm.at[wid, pl.ds(b, 1)], dslot[b], dsem[b])
            pltpu.async_copy(xs_hbm.at[si_v.at[b]], rows[b], rsem[b])

        @pl.loop(0, ch - NBUF, step=NBUF)
        def _(g):
            for b in range(NBUF):
                c = g + b
                pltpu.make_async_copy(xs_hbm.at[si_v.at[c]],
                                      rows[b], rsem[b]).wait()
                pltpu.make_async_copy(dst_hbm.at[wid, pl.ds(c, 1)],
                                      dslot[b], dsem[b]).wait()
                pltpu.async_copy(rows[b], acc_sh.at[dslot[b].at[0]],
                                 ssem[b], add=True)
            for b in range(NBUF):
                c = g + b
                pltpu.make_async_copy(rows[b], acc_sh.at[dslot[b].at[0]],
                                      ssem[b]).wait()
                pltpu.async_copy(dst_hbm.at[wid, pl.ds(c + NBUF, 1)],
                                 dslot[b], dsem[b])
                pltpu.async_copy(xs_hbm.at[si_v.at[c + NBUF]],
                                 rows[b], rsem[b])

        for b in range(NBUF):
            c = ch - NBUF + b
            pltpu.make_async_copy(xs_hbm.at[si_v.at[c]],
                                  rows[b], rsem[b]).wait()
            pltpu.make_async_copy(dst_hbm.at[wid, pl.ds(c, 1)],
                                  dslot[b], dsem[b]).wait()
            pltpu.async_copy(rows[b], acc_sh.at[dslot[b].at[0]],
                             ssem[b], add=True)
        for b in range(NBUF):
            pltpu.make_async_copy(rows[b], acc_sh.at[dslot[b].at[0]],
                                  ssem[b]).wait()

        plsc.subcore_barrier()
        pltpu.sync_copy(acc_sh.at[pl.ds(r0, rpt)],
                        out_hbm.at[cid, pl.ds(r0, rpt)])

    return msg_k


def _dis(d0_ref, d1_ref):
    deg = d0_ref[:, :1] + d1_ref[:, :1] + 1.0
    return lax.rsqrt(deg)


def _tc_scale_body(x_ref, w_ref, d0_ref, d1_ref, o_ref):
    xw = jnp.dot(x_ref[...], w_ref[...], preferred_element_type=jnp.float32)
    o_ref[...] = xw * _dis(d0_ref, d1_ref)


def _ln_relu(t, g_ref, be_ref, eps=1e-5):
    mu = jnp.mean(t, axis=1, keepdims=True)
    var = jnp.mean((t - mu) ** 2, axis=1, keepdims=True)
    t = (t - mu) * lax.rsqrt(var + eps) * g_ref[...] + be_ref[...]
    return jnp.maximum(t, 0.0)


def _tc_mid_body(m0_ref, m1_ref, hs_ref, d0_ref, d1_ref,
                 b_ref, g_ref, be_ref, w2_ref, o_ref):
    dis = _dis(d0_ref, d1_ref)
    t = (m0_ref[...] + m1_ref[...] + hs_ref[...]) * dis + b_ref[...]
    t = _ln_relu(t, g_ref, be_ref)
    o_ref[...] = jnp.dot(t, w2_ref[...],
                         preferred_element_type=jnp.float32) * dis


def _tc_head_body(m0_ref, m1_ref, hs_ref, d0_ref, d1_ref, x_ref,
                  b_ref, g_ref, be_ref, wa_ref, wb_ref, bm1_ref,
                  w2p_ref, bm2p_ref, o_ref):
    dis = _dis(d0_ref, d1_ref)
    t = (m0_ref[...] + m1_ref[...] + hs_ref[...]) * dis + b_ref[...]
    h2 = _ln_relu(t, g_ref, be_ref)
    u = (jnp.dot(h2, wa_ref[...], preferred_element_type=jnp.float32)
         + jnp.dot(x_ref[...], wb_ref[...], preferred_element_type=jnp.float32)
         + bm1_ref[...])
    u = jnp.maximum(u, 0.0)
    o_ref[...] = jnp.dot(u, w2p_ref[...],
                         preferred_element_type=jnp.float32) + bm2p_ref[...]


def kernel(x, edge_index, W1, b1, g1, be1, W2, b2, g2, be2,
           Wm1, bm1, Wm2, bm2):
    n, d = x.shape
    hid = W1.shape[1]
    gout = W2.shape[1]
    mlp_h = Wm1.shape[1]
    out_dim = Wm2.shape[1]

    src3, dst3, ch, ck = _layout_edges(edge_index[0], edge_index[1], n)
    _, npad = _row_split(n)
    ones128 = jnp.ones((ck, 128), jnp.float32)
    zd = jnp.zeros((npad, d), jnp.float32)

    degp = _deg_kernel(n, 128, ch, ck)(dst3, ones128, zd)
    d0, d1 = degp[0], degp[1]

    bn = 1000
    grid = (-(-n // bn),)
    row_spec = lambda w: pl.BlockSpec((bn, w), lambda i: (i, 0))
    full_spec = lambda a, b: pl.BlockSpec((a, b), lambda i: (0, 0))

    msg_k = _msg_kernel(n, d, ch, ck)

    h0s = pl.pallas_call(
        _tc_scale_body,
        grid=grid,
        in_specs=[row_spec(d), full_spec(d, hid), row_spec(128), row_spec(128)],
        out_specs=row_spec(hid),
        out_shape=jax.ShapeDtypeStruct((n, hid), jnp.float32),
    )(x, W1, d0, d1)

    m = msg_k(h0s, src3, dst3, zd)

    h1s = pl.pallas_call(
        _tc_mid_body,
        grid=grid,
        in_specs=[row_spec(hid), row_spec(hid), row_spec(hid),
                  row_spec(128), row_spec(128),
                  full_spec(1, hid), full_spec(1, hid), full_spec(1, hid),
                  full_spec(hid, gout)],
        out_specs=row_spec(gout),
        out_shape=jax.ShapeDtypeStruct((n, gout), jnp.float32),
    )(m[0], m[1], h0s, d0, d1,
      b1.reshape(1, hid), g1.reshape(1, hid), be1.reshape(1, hid), W2)

    m2 = msg_k(h1s, src3, dst3, zd)

    opad = 128
    w2p = jnp.zeros((mlp_h, opad), jnp.float32).at[:, :out_dim].set(Wm2)
    bm2p = jnp.zeros((1, opad), jnp.float32).at[:, :out_dim].set(bm2)
    outp = pl.pallas_call(
        _tc_head_body,
        grid=grid,
        in_specs=[row_spec(gout), row_spec(gout), row_spec(gout),
                  row_spec(128), row_spec(128), row_spec(d),
                  full_spec(1, gout), full_spec(1, gout), full_spec(1, gout),
                  full_spec(gout, mlp_h), full_spec(d, mlp_h),
                  full_spec(1, mlp_h), full_spec(mlp_h, opad),
                  full_spec(1, opad)],
        out_specs=row_spec(opad),
        out_shape=jax.ShapeDtypeStruct((n, opad), jnp.float32),
    )(m2[0], m2[1], h1s, d0, d1, x,
      b2.reshape(1, gout), g2.reshape(1, gout), be2.reshape(1, gout),
      Wm1[:gout], Wm1[gout:], bm1.reshape(1, mlp_h), w2p, bm2p)

    return outp[:, :out_dim]


# speedup vs baseline: 1.2240x; 1.2240x over previous
import functools

import jax
import jax.numpy as jnp
from jax import lax
from jax.experimental import pallas as pl
from jax.experimental.pallas import tpu as pltpu
from jax.experimental.pallas import tpu_sc as plsc

NC, NS = 2, 16
NW = NC * NS


NBUF = 2


def _layout_edges(src, dst, n):
    e = src.shape[0]
    ck = 128
    ch = -(-e // (NW * ck))
    ch = NBUF * (-(-ch // NBUF))
    pad = NW * ch * ck - e
    if pad:
        _, npad = _row_split(n)
        ndump = _acc_rows(n, npad) - n
        i = jnp.arange(pad, dtype=jnp.int32)
        src = jnp.concatenate([src, (i * 131) % n])
        dst = jnp.concatenate([dst, n + i % ndump])
    return src.reshape(NW, ch, ck), dst.reshape(NW, ch, ck), ch, ck


def _row_split(n):
    rpt = 8 * (-(-n // (8 * NS)))
    npad = rpt * NS
    return rpt, npad


def _acc_rows(n, npad):
    return npad if npad > n else npad + 8


def _deg_kernel(n, d, ch, ck):
    rpt, npad = _row_split(n)
    nacc = _acc_rows(n, npad)
    mesh = plsc.VectorSubcoreMesh(core_axis_name="c", subcore_axis_name="s")

    @functools.partial(
        pl.kernel,
        out_type=jax.ShapeDtypeStruct((NC, npad, d), jnp.float32),
        mesh=mesh,
        scratch_types=[
            pltpu.VMEM((ch, ck), jnp.int32),
            pltpu.VMEM((ck, d), jnp.float32),
            pltpu.VMEM_SHARED((nacc, d), jnp.float32),
        ],
    )
    def deg_k(dst_hbm, ones_hbm, z_hbm, out_hbm, idx_v, ones_v, acc_sh):
        cid = lax.axis_index("c")
        sid = lax.axis_index("s")
        wid = sid * NC + cid
        r0 = sid * rpt
        pltpu.sync_copy(z_hbm.at[pl.ds(r0, rpt)], acc_sh.at[pl.ds(r0, rpt)])
        pltpu.sync_copy(dst_hbm.at[wid], idx_v)
        pltpu.sync_copy(ones_hbm, ones_v)
        plsc.subcore_barrier()

        @pl.loop(0, ch)
        def _(c):
            pltpu.sync_copy(ones_v, acc_sh.at[idx_v.at[c]], add=True)

        plsc.subcore_barrier()
        pltpu.sync_copy(acc_sh.at[pl.ds(r0, rpt)],
                        out_hbm.at[cid, pl.ds(r0, rpt)])

    return deg_k


def _msg_kernel(n, d, ch, ck):
    rpt, npad = _row_split(n)
    nacc = _acc_rows(n, npad)
    mesh = plsc.VectorSubcoreMesh(core_axis_name="c", subcore_axis_name="s")

    @functools.partial(
        pl.kernel,
        out_type=jax.ShapeDtypeStruct((NC, npad, d), jnp.float32),
        mesh=mesh,
        scratch_types=[pltpu.VMEM((ch, ck), jnp.int32)]
          + [pltpu.VMEM((ck, d), jnp.float32) for _ in range(NBUF)]
          + [pltpu.VMEM((1, ck), jnp.int32) for _ in range(NBUF)]
          + [pltpu.VMEM_SHARED((nacc, d), jnp.float32)]
          + [pltpu.SemaphoreType.DMA for _ in range(2 * NBUF)],
    )
    def msg_k(xs_hbm, src_hbm, dst_hbm, z_hbm, out_hbm, si_v, *rest):
        rows = rest[:NBUF]
        dslot = rest[NBUF:2 * NBUF]
        acc_sh = rest[2 * NBUF]
        rsem = rest[2 * NBUF + 1:2 * NBUF + 1 + NBUF]
        dsem = rest[2 * NBUF + 1 + NBUF:]
        cid = lax.axis_index("c")
        sid = lax.axis_index("s")
        wid = sid * NC + cid
        r0 = sid * rpt
        pltpu.sync_copy(z_hbm.at[pl.ds(r0, rpt)], acc_sh.at[pl.ds(r0, rpt)])
        pltpu.sync_copy(src_hbm.at[wid], si_v)
        plsc.subcore_barrier()

        for b in range(NBUF):
            pltpu.async_copy(dst_hbm.at[wid, pl.ds(b, 1)], dslot[b], dsem[b])
            pltpu.async_copy(xs_hbm.at[si_v.at[b]], rows[b], rsem[b])

        @pl.loop(0, ch - NBUF, step=NBUF)
        def _(g):
            for b in range(NBUF):
                c = g + b
                pltpu.make_async_copy(xs_hbm.at[si_v.at[c]],
                                      rows[b], rsem[b]).wait()
                pltpu.make_async_copy(dst_hbm.at[wid, pl.ds(c, 1)],
                                      dslot[b], dsem[b]).wait()
                pltpu.sync_copy(rows[b], acc_sh.at[dslot[b].at[0]], add=True)
                pltpu.async_copy(dst_hbm.at[wid, pl.ds(c + NBUF, 1)],
                                 dslot[b], dsem[b])
                pltpu.async_copy(xs_hbm.at[si_v.at[c + NBUF]],
                                 rows[b], rsem[b])

        for b in range(NBUF):
            c = ch - NBUF + b
            pltpu.make_async_copy(xs_hbm.at[si_v.at[c]],
                                  rows[b], rsem[b]).wait()
            pltpu.make_async_copy(dst_hbm.at[wid, pl.ds(c, 1)],
                                  dslot[b], dsem[b]).wait()
            pltpu.sync_copy(rows[b], acc_sh.at[dslot[b].at[0]], add=True)

        plsc.subcore_barrier()
        pltpu.sync_copy(acc_sh.at[pl.ds(r0, rpt)],
                        out_hbm.at[cid, pl.ds(r0, rpt)])

    return msg_k


def _dis(d0_ref, d1_ref):
    deg = d0_ref[:, :1] + d1_ref[:, :1] + 1.0
    return lax.rsqrt(deg)


def _tc_scale_body(x_ref, w_ref, d0_ref, d1_ref, o_ref):
    xw = jnp.dot(x_ref[...], w_ref[...], preferred_element_type=jnp.float32)
    o_ref[...] = xw * _dis(d0_ref, d1_ref)


def _ln_relu(t, g_ref, be_ref, eps=1e-5):
    mu = jnp.mean(t, axis=1, keepdims=True)
    var = jnp.mean((t - mu) ** 2, axis=1, keepdims=True)
    t = (t - mu) * lax.rsqrt(var + eps) * g_ref[...] + be_ref[...]
    return jnp.maximum(t, 0.0)


def _tc_mid_body(m0_ref, m1_ref, hs_ref, d0_ref, d1_ref,
                 b_ref, g_ref, be_ref, w2_ref, o_ref):
    dis = _dis(d0_ref, d1_ref)
    t = (m0_ref[...] + m1_ref[...] + hs_ref[...]) * dis + b_ref[...]
    t = _ln_relu(t, g_ref, be_ref)
    o_ref[...] = jnp.dot(t, w2_ref[...],
                         preferred_element_type=jnp.float32) * dis


def _tc_head_body(m0_ref, m1_ref, hs_ref, d0_ref, d1_ref, x_ref,
                  b_ref, g_ref, be_ref, wa_ref, wb_ref, bm1_ref,
                  w2p_ref, bm2p_ref, o_ref):
    dis = _dis(d0_ref, d1_ref)
    t = (m0_ref[...] + m1_ref[...] + hs_ref[...]) * dis + b_ref[...]
    h2 = _ln_relu(t, g_ref, be_ref)
    u = (jnp.dot(h2, wa_ref[...], preferred_element_type=jnp.float32)
         + jnp.dot(x_ref[...], wb_ref[...], preferred_element_type=jnp.float32)
         + bm1_ref[...])
    u = jnp.maximum(u, 0.0)
    o_ref[...] = jnp.dot(u, w2p_ref[...],
                         preferred_element_type=jnp.float32) + bm2p_ref[...]


def kernel(x, edge_index, W1, b1, g1, be1, W2, b2, g2, be2,
           Wm1, bm1, Wm2, bm2):
    n, d = x.shape
    hid = W1.shape[1]
    gout = W2.shape[1]
    mlp_h = Wm1.shape[1]
    out_dim = Wm2.shape[1]

    src3, dst3, ch, ck = _layout_edges(edge_index[0], edge_index[1], n)
    _, npad = _row_split(n)
    ones128 = jnp.ones((ck, 128), jnp.float32)
    zd = jnp.zeros((npad, d), jnp.float32)

    degp = _deg_kernel(n, 128, ch, ck)(dst3, ones128, zd)
    d0, d1 = degp[0], degp[1]

    bn = 2000
    grid = (-(-n // bn),)
    row_spec = lambda w: pl.BlockSpec((bn, w), lambda i: (i, 0))
    full_spec = lambda a, b: pl.BlockSpec((a, b), lambda i: (0, 0))

    msg_k = _msg_kernel(n, d, ch, ck)

    h0s = pl.pallas_call(
        _tc_scale_body,
        grid=grid,
        in_specs=[row_spec(d), full_spec(d, hid), row_spec(128), row_spec(128)],
        out_specs=row_spec(hid),
        out_shape=jax.ShapeDtypeStruct((n, hid), jnp.float32),
    )(x, W1, d0, d1)

    m = msg_k(h0s, src3, dst3, zd)

    h1s = pl.pallas_call(
        _tc_mid_body,
        grid=grid,
        in_specs=[row_spec(hid), row_spec(hid), row_spec(hid),
                  row_spec(128), row_spec(128),
                  full_spec(1, hid), full_spec(1, hid), full_spec(1, hid),
                  full_spec(hid, gout)],
        out_specs=row_spec(gout),
        out_shape=jax.ShapeDtypeStruct((n, gout), jnp.float32),
    )(m[0], m[1], h0s, d0, d1,
      b1.reshape(1, hid), g1.reshape(1, hid), be1.reshape(1, hid), W2)

    m2 = msg_k(h1s, src3, dst3, zd)

    opad = 128
    w2p = jnp.zeros((mlp_h, opad), jnp.float32).at[:, :out_dim].set(Wm2)
    bm2p = jnp.zeros((1, opad), jnp.float32).at[:, :out_dim].set(bm2)
    outp = pl.pallas_call(
        _tc_head_body,
        grid=grid,
        in_specs=[row_spec(gout), row_spec(gout), row_spec(gout),
                  row_spec(128), row_spec(128), row_spec(d),
                  full_spec(1, gout), full_spec(1, gout), full_spec(1, gout),
                  full_spec(gout, mlp_h), full_spec(d, mlp_h),
                  full_spec(1, mlp_h), full_spec(mlp_h, opad),
                  full_spec(1, opad)],
        out_specs=row_spec(opad),
        out_shape=jax.ShapeDtypeStruct((n, opad), jnp.float32),
    )(m2[0], m2[1], h1s, d0, d1, x,
      b2.reshape(1, gout), g2.reshape(1, gout), be2.reshape(1, gout),
      Wm1[:gout], Wm1[gout:], bm1.reshape(1, mlp_h), w2p, bm2p)

    return outp[:, :out_dim]


# speedup vs baseline: 1.2286x; 1.0038x over previous
import functools

import jax
import jax.numpy as jnp
from jax import lax
from jax.experimental import pallas as pl
from jax.experimental.pallas import tpu as pltpu
from jax.experimental.pallas import tpu_sc as plsc

NC, NS = 2, 16
NW = NC * NS


NBUF = 2


def _layout_edges(src, dst, n):
    e = src.shape[0]
    ck = 128
    ch = -(-e // (NW * ck))
    ch = NBUF * (-(-ch // NBUF))
    pad = NW * ch * ck - e
    if pad:
        _, npad = _row_split(n)
        ndump = _acc_rows(n, npad) - n
        i = jnp.arange(pad, dtype=jnp.int32)
        src = jnp.concatenate([src, (i * 131) % n])
        dst = jnp.concatenate([dst, n + i % ndump])
    return src.reshape(NW, ch, ck), dst.reshape(NW, ch, ck), ch, ck


def _row_split(n):
    rpt = 8 * (-(-n // (8 * NS)))
    npad = rpt * NS
    return rpt, npad


def _acc_rows(n, npad):
    return npad if npad > n else npad + 8


def _deg_kernel(n, d, ch, ck):
    rpt, npad = _row_split(n)
    nacc = _acc_rows(n, npad)
    mesh = plsc.VectorSubcoreMesh(core_axis_name="c", subcore_axis_name="s")

    @functools.partial(
        pl.kernel,
        out_type=jax.ShapeDtypeStruct((NC, npad, d), jnp.float32),
        mesh=mesh,
        scratch_types=[
            pltpu.VMEM((ch, ck), jnp.int32),
            pltpu.VMEM((ck, d), jnp.float32),
            pltpu.VMEM_SHARED((nacc, d), jnp.float32),
        ],
    )
    def deg_k(dst_hbm, ones_hbm, z_hbm, out_hbm, idx_v, ones_v, acc_sh):
        cid = lax.axis_index("c")
        sid = lax.axis_index("s")
        wid = sid * NC + cid
        r0 = sid * rpt
        pltpu.sync_copy(z_hbm.at[pl.ds(r0, rpt)], acc_sh.at[pl.ds(r0, rpt)])
        pltpu.sync_copy(dst_hbm.at[wid], idx_v)
        pltpu.sync_copy(ones_hbm, ones_v)
        plsc.subcore_barrier()

        @pl.loop(0, ch)
        def _(c):
            pltpu.sync_copy(ones_v, acc_sh.at[idx_v.at[c]], add=True)

        plsc.subcore_barrier()
        pltpu.sync_copy(acc_sh.at[pl.ds(r0, rpt)],
                        out_hbm.at[cid, pl.ds(r0, rpt)])

    return deg_k


def _msg_kernel(n, d, ch, ck):
    rpt, npad = _row_split(n)
    nacc = _acc_rows(n, npad)
    mesh = plsc.VectorSubcoreMesh(core_axis_name="c", subcore_axis_name="s")

    @functools.partial(
        pl.kernel,
        out_type=jax.ShapeDtypeStruct((NC, npad, d), jnp.float32),
        mesh=mesh,
        scratch_types=[pltpu.VMEM((ch, ck), jnp.int32)]
          + [pltpu.VMEM((ck, d), jnp.float32) for _ in range(NBUF)]
          + [pltpu.VMEM((1, ck), jnp.int32) for _ in range(NBUF)]
          + [pltpu.VMEM_SHARED((nacc, d), jnp.float32)]
          + [pltpu.SemaphoreType.DMA for _ in range(2 * NBUF)],
    )
    def msg_k(xs_hbm, src_hbm, dst_hbm, z_hbm, out_hbm, si_v, *rest):
        rows = rest[:NBUF]
        dslot = rest[NBUF:2 * NBUF]
        acc_sh = rest[2 * NBUF]
        rsem = rest[2 * NBUF + 1:2 * NBUF + 1 + NBUF]
        dsem = rest[2 * NBUF + 1 + NBUF:]
        cid = lax.axis_index("c")
        sid = lax.axis_index("s")
        wid = sid * NC + cid
        r0 = sid * rpt
        pltpu.sync_copy(z_hbm.at[pl.ds(r0, rpt)], acc_sh.at[pl.ds(r0, rpt)])
        pltpu.sync_copy(src_hbm.at[wid], si_v)
        plsc.subcore_barrier()

        for b in range(NBUF):
            pltpu.async_copy(dst_hbm.at[wid, pl.ds(b, 1)], dslot[b], dsem[b])
            pltpu.async_copy(xs_hbm.at[si_v.at[b]], rows[b], rsem[b])

        @pl.loop(0, ch - NBUF, step=NBUF)
        def _(g):
            for b in range(NBUF):
                c = g + b
                pltpu.make_async_copy(xs_hbm.at[si_v.at[c]],
                                      rows[b], rsem[b]).wait()
                pltpu.make_async_copy(dst_hbm.at[wid, pl.ds(c, 1)],
                                      dslot[b], dsem[b]).wait()
                pltpu.sync_copy(rows[b], acc_sh.at[dslot[b].at[0]], add=True)
                pltpu.async_copy(dst_hbm.at[wid, pl.ds(c + NBUF, 1)],
                                 dslot[b], dsem[b])
                pltpu.async_copy(xs_hbm.at[si_v.at[c + NBUF]],
                                 rows[b], rsem[b])

        for b in range(NBUF):
            c = ch - NBUF + b
            pltpu.make_async_copy(xs_hbm.at[si_v.at[c]],
                                  rows[b], rsem[b]).wait()
            pltpu.make_async_copy(dst_hbm.at[wid, pl.ds(c, 1)],
                                  dslot[b], dsem[b]).wait()
            pltpu.sync_copy(rows[b], acc_sh.at[dslot[b].at[0]], add=True)

        plsc.subcore_barrier()
        pltpu.sync_copy(acc_sh.at[pl.ds(r0, rpt)],
                        out_hbm.at[cid, pl.ds(r0, rpt)])

    return msg_k


def _dis(d0_ref, d1_ref):
    deg = d0_ref[:, :1] + d1_ref[:, :1] + 1.0
    return lax.rsqrt(deg)


def _tc_scale_body(x_ref, w_ref, d0_ref, d1_ref, o_ref):
    xw = jnp.dot(x_ref[...], w_ref[...], preferred_element_type=jnp.float32)
    o_ref[...] = xw * _dis(d0_ref, d1_ref)


def _ln_relu(t, g_ref, be_ref, eps=1e-5):
    mu = jnp.mean(t, axis=1, keepdims=True)
    var = jnp.mean((t - mu) ** 2, axis=1, keepdims=True)
    t = (t - mu) * lax.rsqrt(var + eps) * g_ref[...] + be_ref[...]
    return jnp.maximum(t, 0.0)


def _tc_mid_body(m0_ref, m1_ref, hs_ref, d0_ref, d1_ref,
                 b_ref, g_ref, be_ref, w2_ref, o_ref):
    dis = _dis(d0_ref, d1_ref)
    t = (m0_ref[...] + m1_ref[...] + hs_ref[...]) * dis + b_ref[...]
    t = _ln_relu(t, g_ref, be_ref)
    o_ref[...] = jnp.dot(t, w2_ref[...],
                         preferred_element_type=jnp.float32) * dis


def _tc_head_body(m0_ref, m1_ref, hs_ref, d0_ref, d1_ref, x_ref,
                  b_ref, g_ref, be_ref, wa_ref, wb_ref, bm1_ref,
                  w2p_ref, bm2p_ref, o_ref):
    dis = _dis(d0_ref, d1_ref)
    t = (m0_ref[...] + m1_ref[...] + hs_ref[...]) * dis + b_ref[...]
    h2 = _ln_relu(t, g_ref, be_ref)
    u = (jnp.dot(h2, wa_ref[...], preferred_element_type=jnp.float32)
         + jnp.dot(x_ref[...], wb_ref[...], preferred_element_type=jnp.float32)
         + bm1_ref[...])
    u = jnp.maximum(u, 0.0)
    o_ref[...] = jnp.dot(u, w2p_ref[...],
                         preferred_element_type=jnp.float32) + bm2p_ref[...]


def kernel(x, edge_index, W1, b1, g1, be1, W2, b2, g2, be2,
           Wm1, bm1, Wm2, bm2):
    n, d = x.shape
    hid = W1.shape[1]
    gout = W2.shape[1]
    mlp_h = Wm1.shape[1]
    out_dim = Wm2.shape[1]

    src3, dst3, ch, ck = _layout_edges(edge_index[0], edge_index[1], n)
    _, npad = _row_split(n)
    ones128 = jnp.ones((ck, 128), jnp.float32)
    zd = jnp.zeros((npad, d), jnp.float32)

    degp = _deg_kernel(n, 128, ch, ck)(dst3, ones128, zd)
    d0, d1 = degp[0], degp[1]

    bn = 5000
    grid = (-(-n // bn),)
    row_spec = lambda w: pl.BlockSpec((bn, w), lambda i: (i, 0))
    full_spec = lambda a, b: pl.BlockSpec((a, b), lambda i: (0, 0))

    msg_k = _msg_kernel(n, d, ch, ck)

    h0s = pl.pallas_call(
        _tc_scale_body,
        grid=grid,
        in_specs=[row_spec(d), full_spec(d, hid), row_spec(128), row_spec(128)],
        out_specs=row_spec(hid),
        out_shape=jax.ShapeDtypeStruct((n, hid), jnp.float32),
    )(x, W1, d0, d1)

    m = msg_k(h0s, src3, dst3, zd)

    h1s = pl.pallas_call(
        _tc_mid_body,
        grid=grid,
        in_specs=[row_spec(hid), row_spec(hid), row_spec(hid),
                  row_spec(128), row_spec(128),
                  full_spec(1, hid), full_spec(1, hid), full_spec(1, hid),
                  full_spec(hid, gout)],
        out_specs=row_spec(gout),
        out_shape=jax.ShapeDtypeStruct((n, gout), jnp.float32),
    )(m[0], m[1], h0s, d0, d1,
      b1.reshape(1, hid), g1.reshape(1, hid), be1.reshape(1, hid), W2)

    m2 = msg_k(h1s, src3, dst3, zd)

    opad = 128
    w2p = jnp.zeros((mlp_h, opad), jnp.float32).at[:, :out_dim].set(Wm2)
    bm2p = jnp.zeros((1, opad), jnp.float32).at[:, :out_dim].set(bm2)
    outp = pl.pallas_call(
        _tc_head_body,
        grid=grid,
        in_specs=[row_spec(gout), row_spec(gout), row_spec(gout),
                  row_spec(128), row_spec(128), row_spec(d),
                  full_spec(1, gout), full_spec(1, gout), full_spec(1, gout),
                  full_spec(gout, mlp_h), full_spec(d, mlp_h),
                  full_spec(1, mlp_h), full_spec(mlp_h, opad),
                  full_spec(1, opad)],
        out_specs=row_spec(opad),
        out_shape=jax.ShapeDtypeStruct((n, opad), jnp.float32),
    )(m2[0], m2[1], h1s, d0, d1, x,
      b2.reshape(1, gout), g2.reshape(1, gout), be2.reshape(1, gout),
      Wm1[:gout], Wm1[gout:], bm1.reshape(1, mlp_h), w2p, bm2p)

    return outp[:, :out_dim]
